# baseline (speedup 1.0000x reference)
import functools

import jax
import jax.numpy as jnp
from jax.experimental import pallas as pl


def _mlp3_kernel(x_ref, w1_ref, b1_ref, w2_ref, b2_ref, w3_ref, b3_ref, o_ref):
    x = x_ref[...]
    h = jnp.dot(x, w1_ref[...], preferred_element_type=jnp.float32) + b1_ref[...]
    h = jnp.maximum(h, 0.0)
    h = jnp.dot(h, w2_ref[...], preferred_element_type=jnp.float32) + b2_ref[...]
    h = jnp.maximum(h, 0.0)
    o_ref[...] = jnp.dot(h, w3_ref[...], preferred_element_type=jnp.float32) + b3_ref[...]


def _fused_mlp(x, params, block_rows):
    w1, b1, w2, b2, w3, b3 = params
    rows, d_in = x.shape
    d_h1 = w1.shape[1]
    d_h2 = w2.shape[1]
    d_out = w3.shape[1]
    b1 = b1.reshape(1, d_h1)
    b2 = b2.reshape(1, d_h2)
    b3 = b3.reshape(1, d_out)

    grid = (rows // block_rows,)
    whole = lambda shape: pl.BlockSpec(shape, lambda i: (0, 0))
    return pl.pallas_call(
        _mlp3_kernel,
        grid=grid,
        in_specs=[
            pl.BlockSpec((block_rows, d_in), lambda i: (i, 0)),
            whole(w1.shape),
            whole(b1.shape),
            whole(w2.shape),
            whole(b2.shape),
            whole(w3.shape),
            whole(b3.shape),
        ],
        out_specs=pl.BlockSpec((block_rows, d_out), lambda i: (i, 0)),
        out_shape=jax.ShapeDtypeStruct((rows, d_out), jnp.float32),
    )(x, w1, b1, w2, b2, w3, b3)


@jax.jit
def kernel(nodes, edges, global_attr, node_params, edge_params, global_params):
    new_nodes = _fused_mlp(nodes, node_params, block_rows=1000)
    new_edges = _fused_mlp(edges, edge_params, block_rows=2000)
    g = jnp.pad(global_attr, ((0, 7), (0, 0)))
    new_global = _fused_mlp(g, global_params, block_rows=8)[:1]
    return (new_nodes, new_edges, new_global)


# speedup vs baseline: 1.0721x; 1.0721x over previous
import functools

import jax
import jax.numpy as jnp
from jax.experimental import pallas as pl


def _mlp3_kernel(x_ref, w1_ref, b1_ref, w2_ref, b2_ref, w3_ref, b3_ref, o_ref):
    x = x_ref[...]
    h = jnp.dot(x, w1_ref[...], preferred_element_type=jnp.float32) + b1_ref[...]
    h = jnp.maximum(h, 0.0).astype(jnp.bfloat16)
    h = jnp.dot(h, w2_ref[...], preferred_element_type=jnp.float32) + b2_ref[...]
    h = jnp.maximum(h, 0.0).astype(jnp.bfloat16)
    o_ref[...] = jnp.dot(h, w3_ref[...], preferred_element_type=jnp.float32) + b3_ref[...]


def _fused_mlp(x, params, block_rows):
    w1, b1, w2, b2, w3, b3 = params
    rows, d_in = x.shape
    d_h1 = w1.shape[1]
    d_h2 = w2.shape[1]
    d_out = w3.shape[1]
    x = x.astype(jnp.bfloat16)
    w1 = w1.astype(jnp.bfloat16)
    w2 = w2.astype(jnp.bfloat16)
    w3 = w3.astype(jnp.bfloat16)
    b1 = b1.reshape(1, d_h1)
    b2 = b2.reshape(1, d_h2)
    b3 = b3.reshape(1, d_out)

    grid = (rows // block_rows,)
    whole = lambda shape: pl.BlockSpec(shape, lambda i: (0, 0))
    return pl.pallas_call(
        _mlp3_kernel,
        grid=grid,
        in_specs=[
            pl.BlockSpec((block_rows, d_in), lambda i: (i, 0)),
            whole(w1.shape),
            whole(b1.shape),
            whole(w2.shape),
            whole(b2.shape),
            whole(w3.shape),
            whole(b3.shape),
        ],
        out_specs=pl.BlockSpec((block_rows, d_out), lambda i: (i, 0)),
        out_shape=jax.ShapeDtypeStruct((rows, d_out), jnp.float32),
    )(x, w1, b1, w2, b2, w3, b3)


@jax.jit
def kernel(nodes, edges, global_attr, node_params, edge_params, global_params):
    new_nodes = _fused_mlp(nodes, node_params, block_rows=1000)
    new_edges = _fused_mlp(edges, edge_params, block_rows=2000)
    g = jnp.pad(global_attr, ((0, 7), (0, 0)))
    new_global = _fused_mlp(g, global_params, block_rows=8)[:1]
    return (new_nodes, new_edges, new_global)


# speedup vs baseline: 1.3928x; 1.2991x over previous
import functools

import jax
import jax.numpy as jnp
from jax.experimental import pallas as pl


def _mlp3_kernel(x_ref, w1_ref, b1_ref, w2_ref, b2_ref, w3_ref, b3_ref, o_ref):
    x = x_ref[...]
    h = jnp.dot(x, w1_ref[...], preferred_element_type=jnp.float32) + b1_ref[...]
    h = jnp.maximum(h, 0.0).astype(jnp.bfloat16)
    h = jnp.dot(h, w2_ref[...], preferred_element_type=jnp.float32) + b2_ref[...]
    h = jnp.maximum(h, 0.0).astype(jnp.bfloat16)
    o_ref[...] = jnp.dot(h, w3_ref[...], preferred_element_type=jnp.float32) + b3_ref[...]


def _fused_mlp(x, params, block_rows):
    w1, b1, w2, b2, w3, b3 = params
    rows, d_in = x.shape
    d_h1 = w1.shape[1]
    d_h2 = w2.shape[1]
    d_out = w3.shape[1]
    x = x.astype(jnp.bfloat16)
    w1 = w1.astype(jnp.bfloat16)
    w2 = w2.astype(jnp.bfloat16)
    w3 = w3.astype(jnp.bfloat16)
    b1 = b1.reshape(1, d_h1)
    b2 = b2.reshape(1, d_h2)
    b3 = b3.reshape(1, d_out)

    grid = (rows // block_rows,)
    whole = lambda shape: pl.BlockSpec(shape, lambda i: (0, 0))
    return pl.pallas_call(
        _mlp3_kernel,
        grid=grid,
        in_specs=[
            pl.BlockSpec((block_rows, d_in), lambda i: (i, 0)),
            whole(w1.shape),
            whole(b1.shape),
            whole(w2.shape),
            whole(b2.shape),
            whole(w3.shape),
            whole(b3.shape),
        ],
        out_specs=pl.BlockSpec((block_rows, d_out), lambda i: (i, 0)),
        out_shape=jax.ShapeDtypeStruct((rows, d_out), jnp.float32),
    )(x, w1, b1, w2, b2, w3, b3)


@jax.jit
def kernel(nodes, edges, global_attr, node_params, edge_params, global_params):
    new_nodes = _fused_mlp(nodes, node_params, block_rows=2000)
    new_edges = _fused_mlp(edges, edge_params, block_rows=8000)
    g = jnp.pad(global_attr, ((0, 7), (0, 0)))
    new_global = _fused_mlp(g, global_params, block_rows=8)[:1]
    return (new_nodes, new_edges, new_global)


# speedup vs baseline: 1.4167x; 1.0172x over previous
import functools

import jax
import jax.numpy as jnp
from jax.experimental import pallas as pl


def _mlp3_kernel(x_ref, w1_ref, b1_ref, w2_ref, b2_ref, w3_ref, b3_ref, o_ref):
    x = x_ref[...]
    h = jnp.dot(x, w1_ref[...], preferred_element_type=jnp.float32) + b1_ref[...]
    h = jnp.maximum(h, 0.0).astype(jnp.bfloat16)
    h = jnp.dot(h, w2_ref[...], preferred_element_type=jnp.float32) + b2_ref[...]
    h = jnp.maximum(h, 0.0).astype(jnp.bfloat16)
    o_ref[...] = jnp.dot(h, w3_ref[...], preferred_element_type=jnp.float32) + b3_ref[...]


def _fused_mlp(x, params, block_rows):
    w1, b1, w2, b2, w3, b3 = params
    rows, d_in = x.shape
    d_h1 = w1.shape[1]
    d_h2 = w2.shape[1]
    d_out = w3.shape[1]
    x = x.astype(jnp.bfloat16)
    w1 = w1.astype(jnp.bfloat16)
    w2 = w2.astype(jnp.bfloat16)
    w3 = w3.astype(jnp.bfloat16)
    b1 = b1.reshape(1, d_h1)
    b2 = b2.reshape(1, d_h2)
    b3 = b3.reshape(1, d_out)

    grid = (rows // block_rows,)
    whole = lambda shape: pl.BlockSpec(shape, lambda i: (0, 0))
    return pl.pallas_call(
        _mlp3_kernel,
        grid=grid,
        in_specs=[
            pl.BlockSpec((block_rows, d_in), lambda i: (i, 0)),
            whole(w1.shape),
            whole(b1.shape),
            whole(w2.shape),
            whole(b2.shape),
            whole(w3.shape),
            whole(b3.shape),
        ],
        out_specs=pl.BlockSpec((block_rows, d_out), lambda i: (i, 0)),
        out_shape=jax.ShapeDtypeStruct((rows, d_out), jnp.float32),
    )(x, w1, b1, w2, b2, w3, b3)


@jax.jit
def kernel(nodes, edges, global_attr, node_params, edge_params, global_params):
    new_nodes = _fused_mlp(nodes, node_params, block_rows=5000)
    new_edges = _fused_mlp(edges, edge_params, block_rows=16000)
    g = jnp.pad(global_attr, ((0, 7), (0, 0)))
    new_global = _fused_mlp(g, global_params, block_rows=8)[:1]
    return (new_nodes, new_edges, new_global)


# speedup vs baseline: 1.4181x; 1.0010x over previous
import functools

import jax
import jax.numpy as jnp
from jax.experimental import pallas as pl
from jax.experimental.pallas import tpu as pltpu


def _mlp3_kernel(x_ref, w1_ref, b1_ref, w2_ref, b2_ref, w3_ref, b3_ref, o_ref):
    x = x_ref[...]
    h = jnp.dot(x, w1_ref[...], preferred_element_type=jnp.float32) + b1_ref[...]
    h = jnp.maximum(h, 0.0).astype(jnp.bfloat16)
    h = jnp.dot(h, w2_ref[...], preferred_element_type=jnp.float32) + b2_ref[...]
    h = jnp.maximum(h, 0.0).astype(jnp.bfloat16)
    o_ref[...] = jnp.dot(h, w3_ref[...], preferred_element_type=jnp.float32) + b3_ref[...]


def _fused_mlp(x, params, block_rows):
    w1, b1, w2, b2, w3, b3 = params
    rows, d_in = x.shape
    d_h1 = w1.shape[1]
    d_h2 = w2.shape[1]
    d_out = w3.shape[1]
    x = x.astype(jnp.bfloat16)
    w1 = w1.astype(jnp.bfloat16)
    w2 = w2.astype(jnp.bfloat16)
    w3 = w3.astype(jnp.bfloat16)
    b1 = b1.reshape(1, d_h1)
    b2 = b2.reshape(1, d_h2)
    b3 = b3.reshape(1, d_out)

    grid = (rows // block_rows,)
    whole = lambda shape: pl.BlockSpec(shape, lambda i: (0, 0))
    return pl.pallas_call(
        _mlp3_kernel,
        grid=grid,
        in_specs=[
            pl.BlockSpec((block_rows, d_in), lambda i: (i, 0)),
            whole(w1.shape),
            whole(b1.shape),
            whole(w2.shape),
            whole(b2.shape),
            whole(w3.shape),
            whole(b3.shape),
        ],
        out_specs=pl.BlockSpec((block_rows, d_out), lambda i: (i, 0)),
        out_shape=jax.ShapeDtypeStruct((rows, d_out), jnp.float32),
        compiler_params=pltpu.CompilerParams(
            dimension_semantics=("parallel",),
        ),
    )(x, w1, b1, w2, b2, w3, b3)


@jax.jit
def kernel(nodes, edges, global_attr, node_params, edge_params, global_params):
    new_nodes = _fused_mlp(nodes, node_params, block_rows=5000)
    new_edges = _fused_mlp(edges, edge_params, block_rows=16000)
    g = jnp.pad(global_attr, ((0, 7), (0, 0)))
    new_global = _fused_mlp(g, global_params, block_rows=8)[:1]
    return (new_nodes, new_edges, new_global)
